# Initial kernel scaffold; baseline (speedup 1.0000x reference)
import jax
import jax.numpy as jnp
from jax.experimental import pallas as pl


def kernel(x, edge_index, edge_weight, W1, b1, W2, b2):
    raise NotImplementedError("write your pallas kernel here")


# speedup vs baseline: 35.9306x; 35.9306x over previous
import jax
import jax.numpy as jnp
from jax import lax
from jax.experimental import pallas as pl
from jax.experimental.pallas import tpu as pltpu
from jax.experimental.pallas import tpu_sc as plsc

N = 10000
E = 320000
D_IN = 128
HID = 16
N_CLASSES = 40
PADC = 48

NC = 2
NS = 16
NW = NC * NS
RW = 80
EP = 327680
EPW = EP // NW
NROWS = EP // RW
RPW = NROWS // NW
STR = 640
LASTR = N - (NS - 1) * STR

_mesh = plsc.VectorSubcoreMesh(core_axis_name="c", subcore_axis_name="s")


def _zero_rows(ref, nrows, width):
  zv = jnp.zeros((16,), jnp.float32)

  def body(i, _):
    for k in range(width // 16):
      ref[i, pl.ds(k * 16, 16)] = zv
    return 0

  lax.fori_loop(0, nrows, body, 0, unroll=4)


def _zero_flat(ref, n):
  zv = jnp.zeros((16,), jnp.float32)

  def body(i, _):
    ref[pl.ds(i * 16, 16)] = zv
    return 0

  lax.fori_loop(0, n // 16, body, 0, unroll=4)


def _al8(i):
  return pl.multiple_of(i, 8)


def _deg_body(dst_hbm, ew_hbm, degp_hbm, dst_v, ew_v, zbuf, deg_sh, sem):
  c = lax.axis_index("c")
  s = lax.axis_index("s")
  wid = c * NS + s

  @pl.when(s == 0)
  def _():
    _zero_flat(zbuf, 2000)
    for k in range(N // 2000):
      pltpu.sync_copy(zbuf, deg_sh.at[pl.ds(k * 2000, 2000)])

  plsc.subcore_barrier()

  rb = _al8(wid * RPW)
  pltpu.sync_copy(dst_hbm.at[pl.ds(rb, RPW)], dst_v)
  pltpu.sync_copy(ew_hbm.at[pl.ds(rb, RPW)], ew_v)

  def srow(j, _):
    pltpu.async_copy(ew_v.at[j], deg_sh.at[dst_v.at[j]], sem, add=True).wait()
    return 0

  lax.fori_loop(0, RPW, srow, 0)

  plsc.subcore_barrier()

  @pl.when(s == 0)
  def _():
    pltpu.sync_copy(deg_sh, degp_hbm.at[c, 0])


_deg_call = pl.kernel(
    _deg_body,
    out_type=jax.ShapeDtypeStruct((NC, 1, N), jnp.float32),
    mesh=_mesh,
    compiler_params=pltpu.CompilerParams(use_tc_tiling_on_sc=False),
    scratch_types=[
        pltpu.VMEM((RPW, RW), jnp.int32),
        pltpu.VMEM((RPW, RW), jnp.float32),
        pltpu.VMEM((2000,), jnp.float32),
        pltpu.VMEM_SHARED((N,), jnp.float32),
        pltpu.SemaphoreType.DMA,
    ],
)


def _make_agg(W, G):
  CH = G * RW
  NCHUNK = EPW // CH

  def body(y_hbm, src_hbm, dst_hbm, ew_hbm, aggp_hbm,
           src_v, ew_v, dst_v, rows_v, agg_sh, gsem, ssem):
    c = lax.axis_index("c")
    s = lax.axis_index("s")
    wid = c * NS + s

    _zero_rows(rows_v, STR, W)

    @pl.when(s < NS - 1)
    def _():
      pltpu.sync_copy(rows_v.at[pl.ds(0, STR)],
                      agg_sh.at[pl.ds(_al8(s * STR), STR)])

    @pl.when(s == NS - 1)
    def _():
      pltpu.sync_copy(rows_v.at[pl.ds(0, LASTR)],
                      agg_sh.at[pl.ds((NS - 1) * STR, LASTR)])

    plsc.subcore_barrier()

    for cix in range(NCHUNK):
      ebase = _al8(wid * EPW + cix * CH)
      rbase = _al8(wid * RPW + cix * G)
      pltpu.sync_copy(src_hbm.at[pl.ds(ebase, CH)], src_v)
      pltpu.sync_copy(ew_hbm.at[pl.ds(ebase, CH)], ew_v)
      pltpu.sync_copy(dst_hbm.at[pl.ds(rbase, G)], dst_v)
      pltpu.async_copy(y_hbm.at[src_v], rows_v, gsem).wait()

      def scale(i, _):
        wv = ew_v[pl.ds(i * 16, 16)]
        for u in range(16):
          e = i * 16 + u
          w = wv[u]
          for k in range(W // 16):
            r = rows_v[e, pl.ds(k * 16, 16)]
            rows_v[e, pl.ds(k * 16, 16)] = r * w
        return 0

      lax.fori_loop(0, CH // 16, scale, 0)

      def srow(j, _):
        pltpu.async_copy(
            rows_v.at[pl.ds(j * RW, RW)], agg_sh.at[dst_v.at[j]], ssem,
            add=True).wait()
        return 0

      lax.fori_loop(0, G, srow, 0)

    plsc.subcore_barrier()

    @pl.when(s < NS - 1)
    def _():
      pltpu.sync_copy(agg_sh.at[pl.ds(_al8(s * STR), STR)],
                      aggp_hbm.at[c].at[pl.ds(_al8(s * STR), STR)])

    @pl.when(s == NS - 1)
    def _():
      pltpu.sync_copy(agg_sh.at[pl.ds((NS - 1) * STR, LASTR)],
                      aggp_hbm.at[c].at[pl.ds((NS - 1) * STR, LASTR)])

  return pl.kernel(
      body,
      out_type=jax.ShapeDtypeStruct((NC, N, W), jnp.float32),
      mesh=_mesh,
      compiler_params=pltpu.CompilerParams(use_tc_tiling_on_sc=False),
      scratch_types=[
          pltpu.VMEM((CH,), jnp.int32),
          pltpu.VMEM((CH,), jnp.float32),
          pltpu.VMEM((G, RW), jnp.int32),
          pltpu.VMEM((CH, W), jnp.float32),
          pltpu.VMEM_SHARED((N, W), jnp.float32),
          pltpu.SemaphoreType.DMA,
          pltpu.SemaphoreType.DMA,
      ],
  )


_agg16 = _make_agg(16, 32)
_agg48 = _make_agg(48, 16)


def _mm1_body(x_ref, w_ref, o_ref):
  o_ref[...] = jnp.dot(x_ref[...], w_ref[...],
                       preferred_element_type=jnp.float32)


def _xw1(x, W1):
  return pl.pallas_call(
      _mm1_body,
      out_shape=jax.ShapeDtypeStruct((N, HID), jnp.float32),
  )(x, W1)


def _scale1_body(degp_ref, xw_ref, y_ref, dinv_ref):
  deg = degp_ref[0, 0] + degp_ref[1, 0] + 1.0
  dinv = lax.rsqrt(deg).reshape(N, 1)
  dinv_ref[...] = dinv
  y_ref[...] = dinv * xw_ref[...]


def _scale1(degp, xw1):
  return pl.pallas_call(
      _scale1_body,
      out_shape=(
          jax.ShapeDtypeStruct((N, HID), jnp.float32),
          jax.ShapeDtypeStruct((N, 1), jnp.float32),
      ),
  )(degp, xw1)


def _layer2_body(aggp_ref, xw1_ref, dinv_ref, b1_ref, w2_ref,
                 y2_ref, xw2_ref):
  dinv = dinv_ref[...]
  agg = aggp_ref[0] + aggp_ref[1]
  h = dinv * agg + (dinv * dinv) * xw1_ref[...] + b1_ref[...][None, :]
  h = jnp.maximum(h, 0.0)
  xw2 = jnp.dot(h, w2_ref[...], preferred_element_type=jnp.float32)
  xw2_ref[...] = xw2
  y2 = dinv * xw2
  y2_ref[...] = jnp.concatenate(
      [y2, jnp.zeros((N, PADC - N_CLASSES), jnp.float32)], axis=1)


def _layer2(aggp1, xw1, dinv, b1, W2):
  return pl.pallas_call(
      _layer2_body,
      out_shape=(
          jax.ShapeDtypeStruct((N, PADC), jnp.float32),
          jax.ShapeDtypeStruct((N, N_CLASSES), jnp.float32),
      ),
  )(aggp1, xw1, dinv, b1, W2)


def _final_body(aggp_ref, xw2_ref, dinv_ref, b2_ref, o_ref):
  dinv = dinv_ref[...]
  agg = (aggp_ref[0] + aggp_ref[1])[:, :N_CLASSES]
  pre = dinv * agg + (dinv * dinv) * xw2_ref[...] + b2_ref[...][None, :]
  m = jnp.max(pre, axis=1, keepdims=True)
  lse = jnp.log(jnp.sum(jnp.exp(pre - m), axis=1, keepdims=True)) + m
  o_ref[...] = pre - lse


def _final(aggp2, xw2, dinv, b2):
  return pl.pallas_call(
      _final_body,
      out_shape=jax.ShapeDtypeStruct((N, N_CLASSES), jnp.float32),
  )(aggp2, xw2, dinv, b2)


@jax.jit
def kernel(x, edge_index, edge_weight, W1, b1, W2, b2):
  npad = EP - E
  pad_idx = jnp.arange(npad, dtype=jnp.int32) % N
  src = jnp.concatenate([edge_index[0], pad_idx])
  dst = jnp.concatenate([edge_index[1], pad_idx])
  ew = jnp.concatenate([edge_weight, jnp.zeros((npad,), jnp.float32)])
  dst2d = dst.reshape(NROWS, RW)
  ew2d = ew.reshape(NROWS, RW)

  xw1 = _xw1(x, W1)
  degp = _deg_call(dst2d, ew2d)
  y1, dinv = _scale1(degp, xw1)
  aggp1 = _agg16(y1, src, dst2d, ew)
  y2, xw2 = _layer2(aggp1, xw1, dinv, b1, W2)
  aggp2 = _agg48(y2, src, dst2d, ew)
  return _final(aggp2, xw2, dinv, b2)


# speedup vs baseline: 42.5567x; 1.1844x over previous
import jax
import jax.numpy as jnp
from jax import lax
from jax.experimental import pallas as pl
from jax.experimental.pallas import tpu as pltpu
from jax.experimental.pallas import tpu_sc as plsc

N = 10000
E = 320000
D_IN = 128
HID = 16
N_CLASSES = 40
PADC = 48

NC = 2
NS = 16
NW = NC * NS
RW = 80
EP = 327680
EPW = EP // NW
NROWS = EP // RW
RPW = NROWS // NW
STR = 640
LASTR = N - (NS - 1) * STR

_mesh = plsc.VectorSubcoreMesh(core_axis_name="c", subcore_axis_name="s")


def _zero_rows(ref, nrows, width):
  zv = jnp.zeros((16,), jnp.float32)

  def body(i, _):
    for k in range(width // 16):
      ref[i, pl.ds(k * 16, 16)] = zv
    return 0

  lax.fori_loop(0, nrows, body, 0, unroll=4)


def _zero_flat(ref, n):
  zv = jnp.zeros((16,), jnp.float32)

  def body(i, _):
    ref[pl.ds(i * 16, 16)] = zv
    return 0

  lax.fori_loop(0, n // 16, body, 0, unroll=4)


def _al8(i):
  return pl.multiple_of(i, 8)


def _deg_body(dst_hbm, ew_hbm, degp_hbm, dst_v, ew_v, zbuf, deg_sh, sem):
  c = lax.axis_index("c")
  s = lax.axis_index("s")
  wid = c * NS + s

  @pl.when(s == 0)
  def _():
    _zero_flat(zbuf, 2000)
    for k in range(N // 2000):
      pltpu.sync_copy(zbuf, deg_sh.at[pl.ds(k * 2000, 2000)])

  plsc.subcore_barrier()

  rb = _al8(wid * RPW)
  pltpu.sync_copy(dst_hbm.at[pl.ds(rb, RPW)], dst_v)
  pltpu.sync_copy(ew_hbm.at[pl.ds(rb, RPW)], ew_v)

  def srow(j, _):
    pltpu.async_copy(ew_v.at[j], deg_sh.at[dst_v.at[j]], sem, add=True)
    return 0

  lax.fori_loop(0, RPW, srow, 0)

  def dwait(j, _):
    pltpu.make_async_copy(ew_v.at[0], deg_sh.at[dst_v.at[0]], sem).wait()
    return 0

  lax.fori_loop(0, RPW, dwait, 0)

  plsc.subcore_barrier()

  @pl.when(s == 0)
  def _():
    pltpu.sync_copy(deg_sh, degp_hbm.at[c, 0])


_deg_call = pl.kernel(
    _deg_body,
    out_type=jax.ShapeDtypeStruct((NC, 1, N), jnp.float32),
    mesh=_mesh,
    compiler_params=pltpu.CompilerParams(use_tc_tiling_on_sc=False),
    scratch_types=[
        pltpu.VMEM((RPW, RW), jnp.int32),
        pltpu.VMEM((RPW, RW), jnp.float32),
        pltpu.VMEM((2000,), jnp.float32),
        pltpu.VMEM_SHARED((N,), jnp.float32),
        pltpu.SemaphoreType.DMA,
    ],
)


def _make_agg(W, G):
  CH = G * RW
  NCHUNK = EPW // CH
  NB = 3

  def body(y_hbm, src_hbm, dst_hbm, ew_hbm, aggp_hbm,
           src3, ew3, dst3, rows3, agg_sh, gsem, ssem):
    c = lax.axis_index("c")
    s = lax.axis_index("s")
    wid = c * NS + s

    _zero_rows(rows3.at[0], STR, W)

    @pl.when(s < NS - 1)
    def _():
      pltpu.sync_copy(rows3.at[0].at[pl.ds(0, STR)],
                      agg_sh.at[pl.ds(_al8(s * STR), STR)])

    @pl.when(s == NS - 1)
    def _():
      pltpu.sync_copy(rows3.at[0].at[pl.ds(0, LASTR)],
                      agg_sh.at[pl.ds((NS - 1) * STR, LASTR)])

    plsc.subcore_barrier()

    def stage(cix, b):
      ebase = _al8(wid * EPW + cix * CH)
      rbase = _al8(wid * RPW + cix * G)
      pltpu.sync_copy(src_hbm.at[pl.ds(ebase, CH)], src3.at[b])
      pltpu.sync_copy(ew_hbm.at[pl.ds(ebase, CH)], ew3.at[b])
      pltpu.sync_copy(dst_hbm.at[pl.ds(rbase, G)], dst3.at[b])
      pltpu.async_copy(y_hbm.at[src3.at[b]], rows3.at[b], gsem)

    def gwait(b):
      pltpu.make_async_copy(y_hbm.at[src3.at[b]], rows3.at[b], gsem).wait()

    def scale(b):
      rows = rows3.at[b]
      ew_v = ew3.at[b]

      def sbody(i, _):
        wv = ew_v[pl.ds(i * 16, 16)]
        for u in range(16):
          e = i * 16 + u
          w = wv[u]
          for k in range(W // 16):
            r = rows[e, pl.ds(k * 16, 16)]
            rows[e, pl.ds(k * 16, 16)] = r * w
        return 0

      lax.fori_loop(0, CH // 16, sbody, 0)

    def fire(b):
      def srow(j, _):
        pltpu.async_copy(rows3.at[b].at[pl.ds(j * RW, RW)],
                         agg_sh.at[dst3.at[b].at[j]], ssem, add=True)
        return 0

      lax.fori_loop(0, G, srow, 0)

    def drain():
      def dw(j, _):
        pltpu.make_async_copy(rows3.at[0].at[pl.ds(0, RW)],
                              agg_sh.at[dst3.at[0].at[0]], ssem).wait()
        return 0

      lax.fori_loop(0, G, dw, 0)

    stage(0, 0)
    for cix in range(NCHUNK):
      b = cix % NB
      gwait(b)
      if cix >= 2:
        drain()
      if cix + 1 < NCHUNK:
        stage(cix + 1, (cix + 1) % NB)
      scale(b)
      fire(b)
    drain()
    drain()

    plsc.subcore_barrier()

    @pl.when(s < NS - 1)
    def _():
      pltpu.sync_copy(agg_sh.at[pl.ds(_al8(s * STR), STR)],
                      aggp_hbm.at[c].at[pl.ds(_al8(s * STR), STR)])

    @pl.when(s == NS - 1)
    def _():
      pltpu.sync_copy(agg_sh.at[pl.ds((NS - 1) * STR, LASTR)],
                      aggp_hbm.at[c].at[pl.ds((NS - 1) * STR, LASTR)])

  return pl.kernel(
      body,
      out_type=jax.ShapeDtypeStruct((NC, N, W), jnp.float32),
      mesh=_mesh,
      compiler_params=pltpu.CompilerParams(use_tc_tiling_on_sc=False),
      scratch_types=[
          pltpu.VMEM((NB, CH), jnp.int32),
          pltpu.VMEM((NB, CH), jnp.float32),
          pltpu.VMEM((NB, G, RW), jnp.int32),
          pltpu.VMEM((NB, CH, W), jnp.float32),
          pltpu.VMEM_SHARED((N, W), jnp.float32),
          pltpu.SemaphoreType.DMA,
          pltpu.SemaphoreType.DMA,
      ],
  )


_agg16 = _make_agg(16, 16)
_agg48 = _make_agg(48, 8)


def _mm1_body(x_ref, w_ref, o_ref):
  o_ref[...] = jnp.dot(x_ref[...], w_ref[...],
                       preferred_element_type=jnp.float32)


def _xw1(x, W1):
  return pl.pallas_call(
      _mm1_body,
      out_shape=jax.ShapeDtypeStruct((N, HID), jnp.float32),
  )(x, W1)


def _scale1_body(degp_ref, xw_ref, y_ref, dinv_ref):
  deg = degp_ref[0, 0] + degp_ref[1, 0] + 1.0
  dinv = lax.rsqrt(deg).reshape(N, 1)
  dinv_ref[...] = dinv
  y_ref[...] = dinv * xw_ref[...]


def _scale1(degp, xw1):
  return pl.pallas_call(
      _scale1_body,
      out_shape=(
          jax.ShapeDtypeStruct((N, HID), jnp.float32),
          jax.ShapeDtypeStruct((N, 1), jnp.float32),
      ),
  )(degp, xw1)


def _layer2_body(aggp_ref, xw1_ref, dinv_ref, b1_ref, w2_ref,
                 y2_ref, xw2_ref):
  dinv = dinv_ref[...]
  agg = aggp_ref[0] + aggp_ref[1]
  h = dinv * agg + (dinv * dinv) * xw1_ref[...] + b1_ref[...][None, :]
  h = jnp.maximum(h, 0.0)
  xw2 = jnp.dot(h, w2_ref[...], preferred_element_type=jnp.float32)
  xw2_ref[...] = xw2
  y2 = dinv * xw2
  y2_ref[...] = jnp.concatenate(
      [y2, jnp.zeros((N, PADC - N_CLASSES), jnp.float32)], axis=1)


def _layer2(aggp1, xw1, dinv, b1, W2):
  return pl.pallas_call(
      _layer2_body,
      out_shape=(
          jax.ShapeDtypeStruct((N, PADC), jnp.float32),
          jax.ShapeDtypeStruct((N, N_CLASSES), jnp.float32),
      ),
  )(aggp1, xw1, dinv, b1, W2)


def _final_body(aggp_ref, xw2_ref, dinv_ref, b2_ref, o_ref):
  dinv = dinv_ref[...]
  agg = (aggp_ref[0] + aggp_ref[1])[:, :N_CLASSES]
  pre = dinv * agg + (dinv * dinv) * xw2_ref[...] + b2_ref[...][None, :]
  m = jnp.max(pre, axis=1, keepdims=True)
  lse = jnp.log(jnp.sum(jnp.exp(pre - m), axis=1, keepdims=True)) + m
  o_ref[...] = pre - lse


def _final(aggp2, xw2, dinv, b2):
  return pl.pallas_call(
      _final_body,
      out_shape=jax.ShapeDtypeStruct((N, N_CLASSES), jnp.float32),
  )(aggp2, xw2, dinv, b2)


@jax.jit
def kernel(x, edge_index, edge_weight, W1, b1, W2, b2):
  npad = EP - E
  pad_idx = jnp.arange(npad, dtype=jnp.int32) % N
  src = jnp.concatenate([edge_index[0], pad_idx])
  dst = jnp.concatenate([edge_index[1], pad_idx])
  ew = jnp.concatenate([edge_weight, jnp.zeros((npad,), jnp.float32)])
  dst2d = dst.reshape(NROWS, RW)
  ew2d = ew.reshape(NROWS, RW)

  xw1 = _xw1(x, W1)
  degp = _deg_call(dst2d, ew2d)
  y1, dinv = _scale1(degp, xw1)
  aggp1 = _agg16(y1, src, dst2d, ew)
  y2, xw2 = _layer2(aggp1, xw1, dinv, b1, W2)
  aggp2 = _agg48(y2, src, dst2d, ew)
  return _final(aggp2, xw2, dinv, b2)


# speedup vs baseline: 42.9241x; 1.0086x over previous
import jax
import jax.numpy as jnp
from jax import lax
from jax.experimental import pallas as pl
from jax.experimental.pallas import tpu as pltpu
from jax.experimental.pallas import tpu_sc as plsc

N = 10000
NPAD = 10240
E = 320000
D_IN = 128
HID = 16
N_CLASSES = 40
PADC = 48

NC = 2
NS = 16
NW = NC * NS
RW = 80
EP = 327680
EPW = EP // NW
NROWS = EP // RW
RPW = NROWS // NW
RPT = NROWS // NS
STR = 640
LASTR = N - (NS - 1) * STR
DSTR = NPAD // NS

_mesh = plsc.VectorSubcoreMesh(core_axis_name="c", subcore_axis_name="s")


def _zero_rows(ref, nrows, width):
  zv = jnp.zeros((16,), jnp.float32)

  def body(i, _):
    for k in range(width // 16):
      ref[i, pl.ds(k * 16, 16)] = zv
    return 0

  lax.fori_loop(0, nrows, body, 0, unroll=4)


def _zero_flat(ref, n):
  zv = jnp.zeros((16,), jnp.float32)

  def body(i, _):
    ref[pl.ds(i * 16, 16)] = zv
    return 0

  lax.fori_loop(0, n // 16, body, 0, unroll=4)


def _al8(i):
  return pl.multiple_of(i, 8)


def _rsqrt16(d):
  s = 0.5 * (1.0 + d)
  for _ in range(15):
    s = 0.5 * (s + d / s)
  return 1.0 / s


def _l1_body(xw_hbm, src_hbm, dst2_hbm, ew_hbm,
             deg_hbm, z_hbm, aggp_hbm,
             dstd_v, ewd_v, src_v, ewa_v, dsta_v, dbuf, rows2,
             deg_sh, agg_sh, dsem, gsem, ssem):
  c = lax.axis_index("c")
  s = lax.axis_index("s")
  wid = c * NS + s

  _zero_rows(rows2.at[0], STR, HID)

  @pl.when(s == 0)
  def _():
    _zero_flat(dbuf, 2048)
    for k in range(NPAD // 2048):
      pltpu.sync_copy(dbuf, deg_sh.at[pl.ds(k * 2048, 2048)])

  @pl.when(s < NS - 1)
  def _():
    pltpu.sync_copy(rows2.at[0].at[pl.ds(0, STR)],
                    agg_sh.at[pl.ds(_al8(s * STR), STR)])

  @pl.when(s == NS - 1)
  def _():
    pltpu.sync_copy(rows2.at[0].at[pl.ds(0, LASTR)],
                    agg_sh.at[pl.ds((NS - 1) * STR, LASTR)])

  plsc.subcore_barrier()

  rbd = _al8(s * RPT)
  pltpu.sync_copy(dst2_hbm.at[pl.ds(rbd, RPT)], dstd_v)
  pltpu.sync_copy(ew_hbm.at[pl.ds(_al8(s * RPT * RW), RPT * RW)], ewd_v)

  def dfire(j, _):
    pltpu.async_copy(ewd_v.at[pl.ds(j * RW, RW)],
                     deg_sh.at[dstd_v.at[j]], dsem, add=True)
    return 0

  lax.fori_loop(0, RPT, dfire, 0)

  ebase = _al8(wid * EPW)
  for k in range(8):
    pltpu.sync_copy(src_hbm.at[pl.ds(_al8(ebase + k * 1280), 1280)],
                    src_v.at[k])
  pltpu.sync_copy(ew_hbm.at[pl.ds(ebase, EPW)], ewa_v)
  pltpu.sync_copy(dst2_hbm.at[pl.ds(_al8(wid * RPW), RPW)], dsta_v)

  def ddrain(j, _):
    pltpu.make_async_copy(ewd_v.at[pl.ds(0, RW)],
                          deg_sh.at[dstd_v.at[0]], dsem).wait()
    return 0

  lax.fori_loop(0, RPT, ddrain, 0)
  plsc.subcore_barrier()

  pltpu.sync_copy(deg_sh.at[pl.ds(_al8(s * DSTR), DSTR)],
                  dbuf.at[pl.ds(0, DSTR)])

  def nbody(i, _):
    d = dbuf[pl.ds(i * 16, 16)] + 1.0
    dbuf[pl.ds(i * 16, 16)] = _rsqrt16(d)
    return 0

  lax.fori_loop(0, DSTR // 16, nbody, 0)

  @pl.when((s == 0) & (c == 0))
  def _():
    pltpu.sync_copy(deg_sh.at[pl.ds(0, N)], deg_hbm)

  zv = rows2.at[0]

  @pl.when(s < NS - 1)
  def _():
    pltpu.sync_copy(xw_hbm.at[pl.ds(_al8(s * STR), STR)],
                    zv.at[pl.ds(0, STR)])

  @pl.when(s == NS - 1)
  def _():
    pltpu.sync_copy(xw_hbm.at[pl.ds((NS - 1) * STR, LASTR)],
                    zv.at[pl.ds(0, LASTR)])

  def zbody(i, _):
    wv = dbuf[pl.ds(i * 16, 16)]
    for u in range(16):
      r = i * 16 + u
      w = wv[u]
      zv[r, pl.ds(0, 16)] = zv[r, pl.ds(0, 16)] * w
    return 0

  lax.fori_loop(0, STR // 16, zbody, 0)
  pltpu.sync_copy(zv.at[pl.ds(0, STR)],
                  z_hbm.at[c].at[pl.ds(_al8(s * STR), STR)])

  plsc.subcore_barrier()

  CH = 1280
  G = CH // RW
  NCHUNK = EPW // CH

  def stage(cix, b):
    pltpu.async_copy(z_hbm.at[c].at[src_v.at[cix]], rows2.at[b], gsem)

  def gwait(b):
    pltpu.make_async_copy(z_hbm.at[c].at[src_v.at[0]],
                          rows2.at[b], gsem).wait()

  def scale(b, cix):
    rows = rows2.at[b]

    def sbody(i, _):
      wv = ewa_v[pl.ds(cix * CH + i * 16, 16)]
      for u in range(16):
        e = i * 16 + u
        w = wv[u]
        r = rows[e, pl.ds(0, 16)]
        rows[e, pl.ds(0, 16)] = r * w
      return 0

    lax.fori_loop(0, CH // 16, sbody, 0)

  def fire(b, cix):
    def srow(j, _):
      pltpu.async_copy(rows2.at[b].at[pl.ds(j * RW, RW)],
                       agg_sh.at[dsta_v.at[cix * G + j]], ssem, add=True)
      return 0

    lax.fori_loop(0, G, srow, 0)

  def drain():
    def dw(j, _):
      pltpu.make_async_copy(rows2.at[0].at[pl.ds(0, RW)],
                            agg_sh.at[dsta_v.at[0]], ssem).wait()
      return 0

    lax.fori_loop(0, G, dw, 0)

  stage(0, 0)
  for cix in range(NCHUNK):
    b = cix % 2
    gwait(b)
    if cix >= 1:
      drain()
    if cix + 1 < NCHUNK:
      stage(cix + 1, (cix + 1) % 2)
    scale(b, cix)
    fire(b, cix)
  drain()

  plsc.subcore_barrier()

  @pl.when(s < NS - 1)
  def _():
    pltpu.sync_copy(agg_sh.at[pl.ds(_al8(s * STR), STR)],
                    aggp_hbm.at[c].at[pl.ds(_al8(s * STR), STR)])

  @pl.when(s == NS - 1)
  def _():
    pltpu.sync_copy(agg_sh.at[pl.ds((NS - 1) * STR, LASTR)],
                    aggp_hbm.at[c].at[pl.ds((NS - 1) * STR, LASTR)])


_l1_call = pl.kernel(
    _l1_body,
    out_type=(
        jax.ShapeDtypeStruct((N,), jnp.float32),
        jax.ShapeDtypeStruct((NC, NPAD, HID), jnp.float32),
        jax.ShapeDtypeStruct((NC, N, HID), jnp.float32),
    ),
    mesh=_mesh,
    compiler_params=pltpu.CompilerParams(use_tc_tiling_on_sc=False),
    scratch_types=[
        pltpu.VMEM((RPT, RW), jnp.int32),
        pltpu.VMEM((RPT * RW,), jnp.float32),
        pltpu.VMEM((8, 1280), jnp.int32),
        pltpu.VMEM((EPW,), jnp.float32),
        pltpu.VMEM((RPW, RW), jnp.int32),
        pltpu.VMEM((2048,), jnp.float32),
        pltpu.VMEM((2, 1280, HID), jnp.float32),
        pltpu.VMEM_SHARED((NPAD,), jnp.float32),
        pltpu.VMEM_SHARED((N, HID), jnp.float32),
        pltpu.SemaphoreType.DMA,
        pltpu.SemaphoreType.DMA,
        pltpu.SemaphoreType.DMA,
    ],
)


def _l2_body(y_hbm, src_hbm, dst2_hbm, ew_hbm, aggp_hbm,
             src3, ew3, dst3, rows3, agg_sh, gsem, ssem):
  c = lax.axis_index("c")
  s = lax.axis_index("s")
  wid = c * NS + s
  W = PADC
  CH = 640
  G = CH // RW
  NCHUNK = EPW // CH
  NB = 3

  _zero_rows(rows3.at[0], STR, W)

  @pl.when(s < NS - 1)
  def _():
    pltpu.sync_copy(rows3.at[0].at[pl.ds(0, STR)],
                    agg_sh.at[pl.ds(_al8(s * STR), STR)])

  @pl.when(s == NS - 1)
  def _():
    pltpu.sync_copy(rows3.at[0].at[pl.ds(0, LASTR)],
                    agg_sh.at[pl.ds((NS - 1) * STR, LASTR)])

  plsc.subcore_barrier()

  def stage(cix, b):
    ebase = _al8(wid * EPW + cix * CH)
    rbase = _al8(wid * RPW + cix * G)
    pltpu.sync_copy(src_hbm.at[pl.ds(ebase, CH)], src3.at[b])
    pltpu.sync_copy(ew_hbm.at[pl.ds(ebase, CH)], ew3.at[b])
    pltpu.sync_copy(dst2_hbm.at[pl.ds(rbase, G)], dst3.at[b])
    pltpu.async_copy(y_hbm.at[src3.at[b]], rows3.at[b], gsem)

  def gwait(b):
    pltpu.make_async_copy(y_hbm.at[src3.at[b]], rows3.at[b], gsem).wait()

  def scale(b):
    rows = rows3.at[b]
    ew_v = ew3.at[b]

    def sbody(i, _):
      wv = ew_v[pl.ds(i * 16, 16)]
      for u in range(16):
        e = i * 16 + u
        w = wv[u]
        for k in range(W // 16):
          r = rows[e, pl.ds(k * 16, 16)]
          rows[e, pl.ds(k * 16, 16)] = r * w
      return 0

    lax.fori_loop(0, CH // 16, sbody, 0)

  def fire(b):
    def srow(j, _):
      pltpu.async_copy(rows3.at[b].at[pl.ds(j * RW, RW)],
                       agg_sh.at[dst3.at[b].at[j]], ssem, add=True)
      return 0

    lax.fori_loop(0, G, srow, 0)

  def drain():
    def dw(j, _):
      pltpu.make_async_copy(rows3.at[0].at[pl.ds(0, RW)],
                            agg_sh.at[dst3.at[0].at[0]], ssem).wait()
      return 0

    lax.fori_loop(0, G, dw, 0)

  stage(0, 0)
  for cix in range(NCHUNK):
    b = cix % NB
    gwait(b)
    if cix >= 2:
      drain()
    if cix + 1 < NCHUNK:
      stage(cix + 1, (cix + 1) % NB)
    scale(b)
    fire(b)
  drain()
  drain()

  plsc.subcore_barrier()

  @pl.when(s < NS - 1)
  def _():
    pltpu.sync_copy(agg_sh.at[pl.ds(_al8(s * STR), STR)],
                    aggp_hbm.at[c].at[pl.ds(_al8(s * STR), STR)])

  @pl.when(s == NS - 1)
  def _():
    pltpu.sync_copy(agg_sh.at[pl.ds((NS - 1) * STR, LASTR)],
                    aggp_hbm.at[c].at[pl.ds((NS - 1) * STR, LASTR)])


_l2_call = pl.kernel(
    _l2_body,
    out_type=jax.ShapeDtypeStruct((NC, N, PADC), jnp.float32),
    mesh=_mesh,
    compiler_params=pltpu.CompilerParams(use_tc_tiling_on_sc=False),
    scratch_types=[
        pltpu.VMEM((3, 640), jnp.int32),
        pltpu.VMEM((3, 640), jnp.float32),
        pltpu.VMEM((3, 8, RW), jnp.int32),
        pltpu.VMEM((3, 640, PADC), jnp.float32),
        pltpu.VMEM_SHARED((N, PADC), jnp.float32),
        pltpu.SemaphoreType.DMA,
        pltpu.SemaphoreType.DMA,
    ],
)


def _mm1_body(x_ref, w_ref, o_ref):
  o_ref[...] = jnp.dot(x_ref[...], w_ref[...],
                       preferred_element_type=jnp.float32)


def _xw1(x, W1):
  return pl.pallas_call(
      _mm1_body,
      out_shape=jax.ShapeDtypeStruct((N, HID), jnp.float32),
  )(x, W1)


def _layer2_body(deg_ref, aggp_ref, xw1_ref, b1_ref, w2_ref,
                 y2_ref, xw2_ref, dinv_ref):
  dinv = lax.rsqrt(deg_ref[...] + 1.0).reshape(N, 1)
  dinv_ref[...] = dinv
  agg = aggp_ref[0] + aggp_ref[1]
  h = dinv * agg + (dinv * dinv) * xw1_ref[...] + b1_ref[...][None, :]
  h = jnp.maximum(h, 0.0)
  xw2 = jnp.dot(h, w2_ref[...], preferred_element_type=jnp.float32)
  xw2_ref[...] = xw2
  y2 = dinv * xw2
  y2_ref[...] = jnp.concatenate(
      [y2, jnp.zeros((N, PADC - N_CLASSES), jnp.float32)], axis=1)


def _layer2(deg, aggp1, xw1, b1, W2):
  return pl.pallas_call(
      _layer2_body,
      out_shape=(
          jax.ShapeDtypeStruct((N, PADC), jnp.float32),
          jax.ShapeDtypeStruct((N, N_CLASSES), jnp.float32),
          jax.ShapeDtypeStruct((N, 1), jnp.float32),
      ),
  )(deg, aggp1, xw1, b1, W2)


def _final_body(aggp_ref, xw2_ref, dinv_ref, b2_ref, o_ref):
  dinv = dinv_ref[...]
  agg = (aggp_ref[0] + aggp_ref[1])[:, :N_CLASSES]
  pre = dinv * agg + (dinv * dinv) * xw2_ref[...] + b2_ref[...][None, :]
  m = jnp.max(pre, axis=1, keepdims=True)
  lse = jnp.log(jnp.sum(jnp.exp(pre - m), axis=1, keepdims=True)) + m
  o_ref[...] = pre - lse


def _final(aggp2, xw2, dinv, b2):
  return pl.pallas_call(
      _final_body,
      out_shape=jax.ShapeDtypeStruct((N, N_CLASSES), jnp.float32),
  )(aggp2, xw2, dinv, b2)


@jax.jit
def kernel(x, edge_index, edge_weight, W1, b1, W2, b2):
  npad = EP - E
  pad_idx = jnp.arange(npad, dtype=jnp.int32) % N
  src = jnp.concatenate([edge_index[0], pad_idx])
  dst = jnp.concatenate([edge_index[1], pad_idx])
  ew = jnp.concatenate([edge_weight, jnp.zeros((npad,), jnp.float32)])
  dst2d = dst.reshape(NROWS, RW)

  xw1 = _xw1(x, W1)
  deg, _z, aggp1 = _l1_call(xw1, src, dst2d, ew)
  y2, xw2, dinv = _layer2(deg, aggp1, xw1, b1, W2)
  aggp2 = _l2_call(y2, src, dst2d, ew)
  return _final(aggp2, xw2, dinv, b2)
